# baseline (speedup 1.0000x reference)
import jax
import jax.numpy as jnp
from jax import lax
from jax.experimental import pallas as pl
from jax.experimental.pallas import tpu as pltpu
from jax.experimental.pallas import tpu_sc as plsc

N_USERS = 50000
N_ENTITIES = 100000
N_RELATIONS = 64
EMB_DIM = 64

SC_NC = 2
SC_NS = 16
SC_NW = SC_NC * SC_NS
SC_LANES = 16

GATHER_CHUNK = 128

TC_TILE = 4096

N_BUF = 3


def _sc_gather_body(hidx_hbm, pidx_hbm, nidx_hbm, emb_hbm,
                    h_hbm, p_hbm, n_hbm,
                    idx_v, buf_v, gsems, wsems):
    b = h_hbm.shape[0]
    b_per_w = b // SC_NW
    n_chunks = b_per_w // GATHER_CHUNK
    wid = lax.axis_index("s") * SC_NC + lax.axis_index("c")
    base = wid * b_per_w
    cbase = wid * n_chunks

    units = [(hidx_hbm, emb_hbm, h_hbm), (pidx_hbm, emb_hbm, p_hbm),
             (nidx_hbm, emb_hbm, n_hbm)]

    for u, (idx_hbm, _, _) in enumerate(units):
        pltpu.sync_copy(idx_hbm.at[pl.ds(cbase, n_chunks)], idx_v.at[u])

    def unit_buf(u):
        return buf_v.at[u]

    gh = [
        [
            pltpu.async_copy(
                units[u][1].at[idx_v.at[u].at[j]],
                unit_buf(u).at[pl.ds(j * GATHER_CHUNK, GATHER_CHUNK)],
                gsems[u])
            for j in range(n_chunks)
        ]
        for u in range(len(units))
    ]
    wh = [None] * len(units)
    for u in range(len(units)):
        for h in gh[u]:
            h.wait()
        wh[u] = pltpu.async_copy(
            unit_buf(u), units[u][2].at[pl.ds(base, b_per_w)], wsems[u])
    for u in range(len(units)):
        wh[u].wait()


def _sc_gather(heads2d, pos2d, neg2d, emb16):
    n_chunks_total = heads2d.shape[0]
    b = n_chunks_total * GATHER_CHUNK
    out16 = jax.ShapeDtypeStruct((b, EMB_DIM), jnp.bfloat16)
    b_per_w = b // SC_NW
    n_chunks = b_per_w // GATHER_CHUNK
    run = pl.kernel(
        _sc_gather_body,
        out_type=[out16] * 3,
        mesh=plsc.VectorSubcoreMesh(core_axis_name="c", subcore_axis_name="s"),
        scratch_types=[
            pltpu.VMEM((3, n_chunks, GATHER_CHUNK), jnp.int32),
            pltpu.VMEM((3, b_per_w, EMB_DIM), jnp.bfloat16),
            [pltpu.SemaphoreType.DMA] * 3,
            [pltpu.SemaphoreType.DMA] * 3,
        ],
        compiler_params=pltpu.CompilerParams(use_tc_tiling_on_sc=False),
    )
    return run(heads2d, pos2d, neg2d, emb16)


N_HI = 8
N_LO = N_RELATIONS // N_HI


def _tc_project_body(h_ref, p_ref, n_ref, rel_ref, re_ref, w_ref,
                     ho_ref, ro_ref, po_ref, no_ref):
    rel = rel_ref[...]
    rel64 = jnp.broadcast_to(rel, (rel.shape[0], EMB_DIM))
    hi64 = rel64 // N_LO
    lo64 = rel64 % N_LO
    oh_hi = [hi64 == g for g in range(N_HI)]
    oh_lo = [lo64 == g for g in range(N_LO)]

    w = w_ref[...]

    def project(x_ref, out_ref):
        x = x_ref[...]
        zeros = jnp.zeros_like(x)
        xexp = jnp.concatenate(
            [jnp.where(oh_hi[g], x, zeros) for g in range(N_HI)], axis=1)
        z = jnp.dot(
            xexp, w, preferred_element_type=jnp.float32).astype(jnp.bfloat16)
        zeros_z = jnp.zeros((z.shape[0], EMB_DIM), jnp.bfloat16)
        acc = jnp.where(oh_lo[0], z[:, :EMB_DIM], zeros_z)
        for g in range(1, N_LO):
            acc += jnp.where(
                oh_lo[g], z[:, g * EMB_DIM:(g + 1) * EMB_DIM], zeros_z)
        out_ref[...] = acc.astype(jnp.float32)

    project(h_ref, ho_ref)
    project(p_ref, po_ref)
    project(n_ref, no_ref)

    rel_iota = lax.broadcasted_iota(jnp.int32, (rel.shape[0], N_RELATIONS), 1)
    oh = (rel == rel_iota).astype(jnp.float32)
    ro_ref[...] = jnp.dot(oh, re_ref[...], preferred_element_type=jnp.float32)


def _tc_project(h_rows, p_rows, n_rows, relations, relation_embed, w_flat):
    b = h_rows.shape[0]
    tb = TC_TILE
    grid = (b // tb,)
    row_spec = pl.BlockSpec((tb, EMB_DIM), lambda i: (i, 0))
    idx_spec = pl.BlockSpec((tb, 1), lambda i: (i, 0))
    out = jax.ShapeDtypeStruct((b, EMB_DIM), jnp.float32)
    return pl.pallas_call(
        _tc_project_body,
        grid=grid,
        in_specs=[row_spec] * 3 + [idx_spec] + [
            pl.BlockSpec((N_RELATIONS, EMB_DIM), lambda i: (0, 0)),
            pl.BlockSpec(w_flat.shape, lambda i: (0, 0)),
        ],
        out_specs=[row_spec] * 4,
        out_shape=[out] * 4,
        compiler_params=pltpu.CompilerParams(
            vmem_limit_bytes=100 * 1024 * 1024),
    )(h_rows, p_rows, n_rows, relations.reshape(b, 1),
      relation_embed, w_flat)


def kernel(heads, relations, pos_tails, neg_tails, user_embed, entity_embed,
           relation_embed, trans_W):
    b = heads.shape[0]
    emb16 = jnp.concatenate(
        [user_embed, entity_embed], axis=0).astype(jnp.bfloat16)
    nct = b // GATHER_CHUNK
    h_rows, p_rows, n_rows = _sc_gather(
        heads.reshape(nct, GATHER_CHUNK),
        pos_tails.reshape(nct, GATHER_CHUNK),
        neg_tails.reshape(nct, GATHER_CHUNK),
        emb16)
    kge = trans_W.shape[-1]
    w_flat = trans_W.reshape(N_HI, N_LO, EMB_DIM, kge).transpose(
        0, 2, 1, 3).reshape(N_HI * EMB_DIM, N_LO * kge).astype(jnp.bfloat16)
    h_e, r_e, pos_t_e, neg_t_e = _tc_project(
        h_rows, p_rows, n_rows, relations, relation_embed, w_flat)
    return (h_e, r_e, pos_t_e, neg_t_e)


# speedup vs baseline: 1.0366x; 1.0366x over previous
import jax
import jax.numpy as jnp
from jax import lax
from jax.experimental import pallas as pl
from jax.experimental.pallas import tpu as pltpu
from jax.experimental.pallas import tpu_sc as plsc

N_USERS = 50000
N_ENTITIES = 100000
N_RELATIONS = 64
EMB_DIM = 64

SC_NC = 2
SC_NS = 16
SC_NW = SC_NC * SC_NS
SC_LANES = 16

GATHER_CHUNK = 128

TC_TILE = 2048

N_BUF = 3


def _sc_gather_body(hidx_hbm, pidx_hbm, nidx_hbm, emb_hbm,
                    h_hbm, p_hbm, n_hbm,
                    idx_v, buf_v, gsems, wsems):
    b = h_hbm.shape[0]
    b_per_w = b // SC_NW
    n_chunks = b_per_w // GATHER_CHUNK
    wid = lax.axis_index("s") * SC_NC + lax.axis_index("c")
    base = wid * b_per_w
    cbase = wid * n_chunks

    units = [(hidx_hbm, emb_hbm, h_hbm), (pidx_hbm, emb_hbm, p_hbm),
             (nidx_hbm, emb_hbm, n_hbm)]

    for u, (idx_hbm, _, _) in enumerate(units):
        pltpu.sync_copy(idx_hbm.at[pl.ds(cbase, n_chunks)], idx_v.at[u])

    def unit_buf(u):
        return buf_v.at[u]

    gh = [
        [
            pltpu.async_copy(
                units[u][1].at[idx_v.at[u].at[j]],
                unit_buf(u).at[pl.ds(j * GATHER_CHUNK, GATHER_CHUNK)],
                gsems[u])
            for j in range(n_chunks)
        ]
        for u in range(len(units))
    ]
    wh = [None] * len(units)
    for u in range(len(units)):
        for h in gh[u]:
            h.wait()
        wh[u] = pltpu.async_copy(
            unit_buf(u), units[u][2].at[pl.ds(base, b_per_w)], wsems[u])
    for u in range(len(units)):
        wh[u].wait()


def _sc_gather(heads2d, pos2d, neg2d, emb16):
    n_chunks_total = heads2d.shape[0]
    b = n_chunks_total * GATHER_CHUNK
    out16 = jax.ShapeDtypeStruct((b, EMB_DIM), jnp.bfloat16)
    b_per_w = b // SC_NW
    n_chunks = b_per_w // GATHER_CHUNK
    run = pl.kernel(
        _sc_gather_body,
        out_type=[out16] * 3,
        mesh=plsc.VectorSubcoreMesh(core_axis_name="c", subcore_axis_name="s"),
        scratch_types=[
            pltpu.VMEM((3, n_chunks, GATHER_CHUNK), jnp.int32),
            pltpu.VMEM((3, b_per_w, EMB_DIM), jnp.bfloat16),
            [pltpu.SemaphoreType.DMA] * 3,
            [pltpu.SemaphoreType.DMA] * 3,
        ],
        compiler_params=pltpu.CompilerParams(use_tc_tiling_on_sc=False),
    )
    return run(heads2d, pos2d, neg2d, emb16)


N_HI = 8
N_LO = N_RELATIONS // N_HI


def _tc_project_body(h_ref, p_ref, n_ref, rel_ref, re_ref, w_ref,
                     ho_ref, ro_ref, po_ref, no_ref):
    rel = rel_ref[...]
    rel64 = jnp.broadcast_to(rel, (rel.shape[0], EMB_DIM))
    hi64 = rel64 // N_LO
    lo64 = rel64 % N_LO
    oh_hi = [hi64 == g for g in range(N_HI)]
    oh_lo = [lo64 == g for g in range(N_LO)]

    w = w_ref[...]

    def project(x_ref, out_ref):
        x = x_ref[...]
        zeros = jnp.zeros_like(x)
        xexp = jnp.concatenate(
            [jnp.where(oh_hi[g], x, zeros) for g in range(N_HI)], axis=1)
        z = jnp.dot(
            xexp, w, preferred_element_type=jnp.float32).astype(jnp.bfloat16)
        zeros_z = jnp.zeros((z.shape[0], EMB_DIM), jnp.bfloat16)
        acc = jnp.where(oh_lo[0], z[:, :EMB_DIM], zeros_z)
        for g in range(1, N_LO):
            acc += jnp.where(
                oh_lo[g], z[:, g * EMB_DIM:(g + 1) * EMB_DIM], zeros_z)
        out_ref[...] = acc.astype(jnp.float32)

    project(h_ref, ho_ref)
    project(p_ref, po_ref)
    project(n_ref, no_ref)

    rel_iota = lax.broadcasted_iota(jnp.int32, (rel.shape[0], N_RELATIONS), 1)
    oh = (rel == rel_iota).astype(jnp.float32)
    ro_ref[...] = jnp.dot(oh, re_ref[...], preferred_element_type=jnp.float32)


def _tc_project(h_rows, p_rows, n_rows, relations, relation_embed, w_flat):
    b = h_rows.shape[0]
    tb = TC_TILE
    grid = (b // tb,)
    row_spec = pl.BlockSpec((tb, EMB_DIM), lambda i: (i, 0))
    idx_spec = pl.BlockSpec((tb, 1), lambda i: (i, 0))
    out = jax.ShapeDtypeStruct((b, EMB_DIM), jnp.float32)
    return pl.pallas_call(
        _tc_project_body,
        grid=grid,
        in_specs=[row_spec] * 3 + [idx_spec] + [
            pl.BlockSpec((N_RELATIONS, EMB_DIM), lambda i: (0, 0)),
            pl.BlockSpec(w_flat.shape, lambda i: (0, 0)),
        ],
        out_specs=[row_spec] * 4,
        out_shape=[out] * 4,
        compiler_params=pltpu.CompilerParams(
            vmem_limit_bytes=100 * 1024 * 1024),
    )(h_rows, p_rows, n_rows, relations.reshape(b, 1),
      relation_embed, w_flat)


def kernel(heads, relations, pos_tails, neg_tails, user_embed, entity_embed,
           relation_embed, trans_W):
    b = heads.shape[0]
    emb16 = jnp.concatenate(
        [user_embed, entity_embed], axis=0).astype(jnp.bfloat16)
    nct = b // GATHER_CHUNK
    h_rows, p_rows, n_rows = _sc_gather(
        heads.reshape(nct, GATHER_CHUNK),
        pos_tails.reshape(nct, GATHER_CHUNK),
        neg_tails.reshape(nct, GATHER_CHUNK),
        emb16)
    kge = trans_W.shape[-1]
    w_flat = trans_W.reshape(N_HI, N_LO, EMB_DIM, kge).transpose(
        0, 2, 1, 3).reshape(N_HI * EMB_DIM, N_LO * kge).astype(jnp.bfloat16)
    h_e, r_e, pos_t_e, neg_t_e = _tc_project(
        h_rows, p_rows, n_rows, relations, relation_embed, w_flat)
    return (h_e, r_e, pos_t_e, neg_t_e)
